# baseline (speedup 1.0000x reference)
import functools

import jax
import jax.numpy as jnp
from jax.experimental import pallas as pl
from jax.experimental.pallas import tpu as pltpu


def _mp_kernel(ts_ref, concept_ref, w_ref, er_ref, ei_ref,
               out_ref, *, n_steps, bn, half):
    concept = concept_ref[...]
    w_pair = w_ref[...].reshape(2 * bn, concept.shape[1])
    dims = (((1,), (1,)), ((), ()))
    c_pair = jax.lax.dot_general(concept, w_pair, dims,
                                 preferred_element_type=jnp.float32)
    cr, ci = c_pair[:, :bn], c_pair[:, bn:]

    col = pl.ds(pl.program_id(0) * bn, bn)
    e1r, e2r, e3r = er_ref[0:1, col], er_ref[1:2, col], er_ref[2:3, col]
    e1i, e2i, e3i = ei_ref[0:1, col], ei_ref[1:2, col], ei_ref[2:3, col]

    root_r, root_i = cr[0:1, :], ci[0:1, :]
    dom_r = cr[1:6, :] + (root_r * e1r - root_i * e1i)
    dom_i = ci[1:6, :] + (root_r * e1i + root_i * e1r)

    fr, fi = cr[6:18, :], ci[6:18, :]
    fr_new, fi_new = fr, fi
    rows12 = jax.lax.broadcasted_iota(jnp.int32, fr.shape, 0)
    s = jnp.zeros((), jnp.int32)
    for i in range(n_steps):
        ur = dom_r[i:i + 1, :] * e2r - dom_i[i:i + 1, :] * e2i
        ui = dom_r[i:i + 1, :] * e2i + dom_i[i:i + 1, :] * e2r
        mask = rows12 == jnp.clip(s, 0, fr.shape[0] - 1)
        fr_new = jnp.where(mask, fr + ur, fr_new)
        fi_new = jnp.where(mask, fi + ui, fi_new)
        s = s + ts_ref[2, i]

    ir, ii = cr[18:54, :], ci[18:54, :]
    ir_new, ii_new = ir, ii
    rows36 = jax.lax.broadcasted_iota(jnp.int32, ir.shape, 0)
    s = jnp.zeros((), jnp.int32)
    for i in range(n_steps):
        ur = fr_new[i:i + 1, :] * e3r - fi_new[i:i + 1, :] * e3i
        ui = fr_new[i:i + 1, :] * e3i + fi_new[i:i + 1, :] * e3r
        mask = rows36 == jnp.clip(s, 0, ir.shape[0] - 1)
        ir_new = jnp.where(mask, ir + ur, ir_new)
        ii_new = jnp.where(mask, ii + ui, ii_new)
        s = s + ts_ref[3, i]

    icol = pl.ds(half + pl.program_id(0) * bn, bn)
    out_ref[:, col] = jnp.concatenate(
        [cr[0:1, :], dom_r / 2, fr_new / 3, ir_new / 4], axis=0)
    out_ref[:, icol] = jnp.concatenate(
        [ci[0:1, :], dom_i / 2, fi_new / 3, ii_new / 4], axis=0)


def kernel(concept_embed, edge_real, edge_imag, tree_structure, W):
    n, h = concept_embed.shape
    half = h // 2
    bn = 128
    nj = half // bn
    w3 = W.reshape(2, half, h)

    out = pl.pallas_call(
        functools.partial(_mp_kernel, n_steps=tree_structure.shape[1],
                          bn=bn, half=half),
        grid=(nj,),
        in_specs=[
            pl.BlockSpec(memory_space=pltpu.SMEM),
            pl.BlockSpec((n, h), lambda j: (0, 0)),
            pl.BlockSpec((2, bn, h), lambda j: (0, j, 0)),
            pl.BlockSpec((3, half), lambda j: (0, 0)),
            pl.BlockSpec((3, half), lambda j: (0, 0)),
        ],
        out_specs=pl.BlockSpec((n, h), lambda j: (0, 0)),
        out_shape=jax.ShapeDtypeStruct((n, h), jnp.float32),
        compiler_params=pltpu.CompilerParams(
            dimension_semantics=("parallel",)),
    )(tree_structure, concept_embed, w3, edge_real, edge_imag)
    return out


# speedup vs baseline: 1.1616x; 1.1616x over previous
import functools

import jax
import jax.numpy as jnp
from jax.experimental import pallas as pl
from jax.experimental.pallas import tpu as pltpu


def _mp_kernel(ts_ref, concept_ref, w_ref, er_ref, ei_ref,
               out_ref, *, n_steps, bn, half):
    concept = concept_ref[...]
    w_pair = w_ref[...].reshape(2 * bn, concept.shape[1])
    dims = (((1,), (1,)), ((), ()))
    c_pair = jax.lax.dot_general(concept, w_pair, dims,
                                 preferred_element_type=jnp.float32)
    cr, ci = c_pair[:, :bn], c_pair[:, bn:]

    col = pl.ds(pl.program_id(0) * bn, bn)
    e1r, e2r, e3r = er_ref[0:1, col], er_ref[1:2, col], er_ref[2:3, col]
    e1i, e2i, e3i = ei_ref[0:1, col], ei_ref[1:2, col], ei_ref[2:3, col]

    root_r, root_i = cr[0:1, :], ci[0:1, :]
    dom_r = cr[1:6, :] + (root_r * e1r - root_i * e1i)
    dom_i = ci[1:6, :] + (root_r * e1i + root_i * e1r)

    fr, fi = cr[6:18, :], ci[6:18, :]
    fr_new, fi_new = fr, fi
    rows12 = jax.lax.broadcasted_iota(jnp.int32, fr.shape, 0)
    s = jnp.zeros((), jnp.int32)
    for i in range(n_steps):
        ur = dom_r[i:i + 1, :] * e2r - dom_i[i:i + 1, :] * e2i
        ui = dom_r[i:i + 1, :] * e2i + dom_i[i:i + 1, :] * e2r
        mask = rows12 == jnp.clip(s, 0, fr.shape[0] - 1)
        fr_new = jnp.where(mask, fr + ur, fr_new)
        fi_new = jnp.where(mask, fi + ui, fi_new)
        s = s + ts_ref[2, i]

    ir, ii = cr[18:54, :], ci[18:54, :]
    ir_new, ii_new = ir, ii
    rows36 = jax.lax.broadcasted_iota(jnp.int32, ir.shape, 0)
    s = jnp.zeros((), jnp.int32)
    for i in range(n_steps):
        ur = fr_new[i:i + 1, :] * e3r - fi_new[i:i + 1, :] * e3i
        ui = fr_new[i:i + 1, :] * e3i + fi_new[i:i + 1, :] * e3r
        mask = rows36 == jnp.clip(s, 0, ir.shape[0] - 1)
        ir_new = jnp.where(mask, ir + ur, ir_new)
        ii_new = jnp.where(mask, ii + ui, ii_new)
        s = s + ts_ref[3, i]

    icol = pl.ds(half + pl.program_id(0) * bn, bn)
    out_ref[:, col] = jnp.concatenate(
        [cr[0:1, :], dom_r / 2, fr_new / 3, ir_new / 4], axis=0)
    out_ref[:, icol] = jnp.concatenate(
        [ci[0:1, :], dom_i / 2, fi_new / 3, ii_new / 4], axis=0)


def kernel(concept_embed, edge_real, edge_imag, tree_structure, W):
    n, h = concept_embed.shape
    half = h // 2
    bn = 256
    nj = half // bn
    w3 = W.reshape(2, half, h)

    out = pl.pallas_call(
        functools.partial(_mp_kernel, n_steps=tree_structure.shape[1],
                          bn=bn, half=half),
        grid=(nj,),
        in_specs=[
            pl.BlockSpec(memory_space=pltpu.SMEM),
            pl.BlockSpec((n, h), lambda j: (0, 0)),
            pl.BlockSpec((2, bn, h), lambda j: (0, j, 0)),
            pl.BlockSpec((3, half), lambda j: (0, 0)),
            pl.BlockSpec((3, half), lambda j: (0, 0)),
        ],
        out_specs=pl.BlockSpec((n, h), lambda j: (0, 0)),
        out_shape=jax.ShapeDtypeStruct((n, h), jnp.float32),
        compiler_params=pltpu.CompilerParams(
            dimension_semantics=("parallel",)),
    )(tree_structure, concept_embed, w3, edge_real, edge_imag)
    return out
